# baseline (speedup 1.0000x reference)
import functools

import jax
import jax.numpy as jnp
from jax import lax
from jax.experimental import pallas as pl
from jax.experimental.pallas import tpu as pltpu
from jax.experimental.pallas import tpu_sc as plsc

N_NODES = 10000
IN_FEAT = 128
HIDDEN = 128
N_CLASSES = 64
N_EDGES = 320000

NC = 2
NS = 16
NW = NC * NS
EPB = 120
NB = 84
PW = NB * EPB
E_PAD = NW * PW
N_PAD = 10240
SLICE = N_PAD // NS
N_ACC = 10112
SLICE_A = N_ACC // NS
N_DUMMY = N_ACC - N_NODES

_mesh = plsc.VectorSubcoreMesh(core_axis_name="c", subcore_axis_name="s")


@functools.partial(
    pl.kernel,
    out_type=jax.ShapeDtypeStruct((NC, N_PAD), jnp.float32),
    mesh=_mesh,
    scratch_types=[
        pltpu.VMEM((NB, EPB), jnp.int32),
        pltpu.VMEM((EPB,), jnp.float32),
        pltpu.VMEM((SLICE,), jnp.float32),
        pltpu.VMEM_SHARED((N_PAD,), jnp.float32),
        pltpu.SemaphoreType.DMA,
    ],
)
def _deg_kernel(dst_hbm, out_hbm, idx_v, ones_v, z_v, acc, sem):
    c = lax.axis_index("c")
    s = lax.axis_index("s")
    wid = c * NS + s
    for i in range(EPB // 16):
        ones_v[pl.ds(i * 16, 16)] = jnp.full((16,), 1.0, jnp.float32)
    if EPB % 16:
        ones_v[pl.ds(EPB - 16, 16)] = jnp.full((16,), 1.0, jnp.float32)
    for i in range(SLICE // 16):
        z_v[pl.ds(i * 16, 16)] = jnp.zeros((16,), jnp.float32)
    pltpu.sync_copy(z_v, acc.at[pl.ds(s * SLICE, SLICE)])
    pltpu.async_copy(dst_hbm.at[wid], idx_v, sem).wait()
    plsc.subcore_barrier()
    descs = [pltpu.async_copy(ones_v, acc.at[idx_v.at[j]], sem, add=True)
             for j in range(NB)]
    for d in descs:
        d.wait()
    plsc.subcore_barrier()
    pltpu.sync_copy(acc.at[pl.ds(s * SLICE, SLICE)],
                    out_hbm.at[c, pl.ds(s * SLICE, SLICE)])


def _make_prop(D):
    @functools.partial(
        pl.kernel,
        out_type=jax.ShapeDtypeStruct((NC, N_ACC, D), jnp.float32),
        mesh=_mesh,
        scratch_types=[
            pltpu.VMEM((4, 2, EPB), jnp.int32),
            pltpu.VMEM((3, EPB, D), jnp.float32),
            pltpu.VMEM_SHARED((N_ACC, D), jnp.float32),
            [pltpu.SemaphoreType.DMA for _ in range(4)],
            [pltpu.SemaphoreType.DMA for _ in range(3)],
            [pltpu.SemaphoreType.DMA for _ in range(3)],
            pltpu.SemaphoreType.DMA,
        ],
    )
    def _prop(h_hbm, eidx_hbm, out_hbm, eidx, rows, acc,
              isems, gsems, ssems, csem):
        c = lax.axis_index("c")
        s = lax.axis_index("s")
        wid = c * NS + s
        pltpu.async_copy(h_hbm.at[pl.ds(s * SLICE_A, SLICE_A)],
                         acc.at[pl.ds(s * SLICE_A, SLICE_A)], csem).wait()
        plsc.subcore_barrier()

        def iload(t):
            return pltpu.async_copy(eidx_hbm.at[wid, t], eidx.at[t % 4],
                                    isems[t % 4])

        def gat(t):
            return pltpu.async_copy(h_hbm.at[eidx.at[t % 4, 0]],
                                    rows.at[t % 3], gsems[t % 3])

        def sca(t):
            return pltpu.async_copy(rows.at[t % 3],
                                    acc.at[eidx.at[t % 4, 1]],
                                    ssems[t % 3], add=True)

        idd = {t: iload(t) for t in range(min(3, NB))}
        gd = {}
        for t in range(min(2, NB)):
            idd.pop(t).wait()
            gd[t] = gat(t)
        sd = {}
        for t in range(NB):
            gd.pop(t).wait()
            sd[t] = sca(t)
            k = t + 2
            if k < NB:
                if t - 1 in sd:
                    sd.pop(t - 1).wait()
                kk = k + 1
                if kk < NB:
                    idd[kk] = iload(kk)
                if k in idd:
                    idd.pop(k).wait()
                gd[k] = gat(k)
        for t in sorted(sd):
            sd.pop(t).wait()
        plsc.subcore_barrier()
        pltpu.sync_copy(acc.at[pl.ds(s * SLICE_A, SLICE_A)],
                        out_hbm.at[c, pl.ds(s * SLICE_A, SLICE_A)])

    return _prop


_prop128 = _make_prop(HIDDEN)


_R = 512
_G = N_PAD // _R


def _k1a_body(x_ref, w_ref, h_ref):
    h_ref[...] = jnp.dot(x_ref[...], w_ref[...],
                         preferred_element_type=jnp.float32)


def _k1b_body(p0_ref, p1_ref, h_ref, hs_ref, dis_ref):
    deg = p0_ref[...] + p1_ref[...] + 1.0
    dis = lax.rsqrt(deg)
    hs_ref[...] = dis * h_ref[...]
    dis_ref[...] = dis


def _k2_body(q0_ref, q1_ref, hp_ref, dis_ref, w_ref, b_ref, out_ref):
    dis = dis_ref[...]
    pre = dis * (q0_ref[...] + q1_ref[...] - hp_ref[...]) + b_ref[...]
    h1 = jnp.maximum(pre, 0.0)
    out_ref[...] = dis * jnp.dot(h1, w_ref[...],
                                 preferred_element_type=jnp.float32)


def _k3_body(q0_ref, q1_ref, hp_ref, dis_ref, b_ref, out_ref):
    o128 = dis_ref[...] * (q0_ref[...] + q1_ref[...] - hp_ref[...])
    o = o128[:, :N_CLASSES] + b_ref[...]
    m = jnp.max(o, axis=1, keepdims=True)
    e = o - m
    lse = jnp.log(jnp.sum(jnp.exp(e), axis=1, keepdims=True))
    out_ref[...] = e - lse


def _col_spec():
    return pl.BlockSpec((_R, 1), lambda i: (i, 0))


def _row_spec(d):
    return pl.BlockSpec((_R, d), lambda i: (i, 0))


def _full_spec(r, d):
    return pl.BlockSpec((r, d), lambda i: (0, 0))


def kernel(x, edge_index, W1, b1, W2, b2):
    src = edge_index[0].astype(jnp.int32)
    dst = edge_index[1].astype(jnp.int32)
    pad_idx = N_NODES + (jnp.arange(E_PAD - N_EDGES, dtype=jnp.int32)
                         % N_DUMMY)
    src3 = jnp.concatenate([src, pad_idx]).reshape(NW, NB, EPB)
    dst3 = jnp.concatenate([dst, pad_idx]).reshape(NW, NB, EPB)
    eidx3 = jnp.stack([src3, dst3], axis=2)
    x_pad = jnp.zeros((N_PAD, IN_FEAT), jnp.float32).at[:N_NODES].set(x)
    b1r = b1.reshape(1, HIDDEN)
    b2r = b2.reshape(1, N_CLASSES)

    degp = _deg_kernel(dst3)
    p0 = degp[0][:, None]
    p1 = degp[1][:, None]

    h_raw = pl.pallas_call(
        _k1a_body,
        grid=(_G,),
        in_specs=[_row_spec(IN_FEAT), _full_spec(IN_FEAT, HIDDEN)],
        out_specs=_row_spec(HIDDEN),
        out_shape=jax.ShapeDtypeStruct((N_PAD, HIDDEN), jnp.float32),
    )(x_pad, W1)

    h1p, dis = pl.pallas_call(
        _k1b_body,
        grid=(_G,),
        in_specs=[_col_spec(), _col_spec(), _row_spec(HIDDEN)],
        out_specs=[_row_spec(HIDDEN), _col_spec()],
        out_shape=[jax.ShapeDtypeStruct((N_PAD, HIDDEN), jnp.float32),
                   jax.ShapeDtypeStruct((N_PAD, 1), jnp.float32)],
    )(p0, p1, h_raw)

    agg1 = _prop128(h1p, eidx3)

    W2p = jnp.zeros((HIDDEN, HIDDEN), jnp.float32).at[:, :N_CLASSES].set(W2)

    h2p = pl.pallas_call(
        _k2_body,
        grid=(_G,),
        in_specs=[_row_spec(HIDDEN), _row_spec(HIDDEN), _row_spec(HIDDEN),
                  _col_spec(), _full_spec(HIDDEN, HIDDEN),
                  _full_spec(1, HIDDEN)],
        out_specs=_row_spec(HIDDEN),
        out_shape=jax.ShapeDtypeStruct((N_PAD, HIDDEN), jnp.float32),
    )(agg1[0], agg1[1], h1p, dis, W2p, b1r)

    agg2 = _prop128(h2p, eidx3)

    out = pl.pallas_call(
        _k3_body,
        grid=(_G,),
        in_specs=[_row_spec(HIDDEN), _row_spec(HIDDEN),
                  _row_spec(HIDDEN), _col_spec(),
                  _full_spec(1, N_CLASSES)],
        out_specs=_row_spec(N_CLASSES),
        out_shape=jax.ShapeDtypeStruct((N_NODES, N_CLASSES), jnp.float32),
    )(agg2[0], agg2[1], h2p, dis, b2r)

    return out


# speedup vs baseline: 1.0206x; 1.0206x over previous
import functools

import jax
import jax.numpy as jnp
from jax import lax
from jax.experimental import pallas as pl
from jax.experimental.pallas import tpu as pltpu
from jax.experimental.pallas import tpu_sc as plsc

N_NODES = 10000
IN_FEAT = 128
HIDDEN = 128
N_CLASSES = 64
N_EDGES = 320000

NC = 2
NS = 16
NW = NC * NS
EPB = 120
NB = 84
PW = NB * EPB
E_PAD = NW * PW
N_PAD = 10240
SLICE = N_PAD // NS
N_ACC = 10112
SLICE_A = N_ACC // NS
N_DUMMY = N_ACC - N_NODES

_mesh = plsc.VectorSubcoreMesh(core_axis_name="c", subcore_axis_name="s")


@functools.partial(
    pl.kernel,
    out_type=jax.ShapeDtypeStruct((NC, N_PAD), jnp.float32),
    mesh=_mesh,
    scratch_types=[
        pltpu.VMEM((NB, EPB), jnp.int32),
        pltpu.VMEM((EPB,), jnp.float32),
        pltpu.VMEM((SLICE,), jnp.float32),
        pltpu.VMEM_SHARED((N_PAD,), jnp.float32),
        pltpu.SemaphoreType.DMA,
    ],
)
def _deg_kernel(dst_hbm, out_hbm, idx_v, ones_v, z_v, acc, sem):
    c = lax.axis_index("c")
    s = lax.axis_index("s")
    wid = c * NS + s
    for i in range(EPB // 16):
        ones_v[pl.ds(i * 16, 16)] = jnp.full((16,), 1.0, jnp.float32)
    if EPB % 16:
        ones_v[pl.ds(EPB - 16, 16)] = jnp.full((16,), 1.0, jnp.float32)
    for i in range(SLICE // 16):
        z_v[pl.ds(i * 16, 16)] = jnp.zeros((16,), jnp.float32)
    pltpu.sync_copy(z_v, acc.at[pl.ds(s * SLICE, SLICE)])
    pltpu.async_copy(dst_hbm.at[wid], idx_v, sem).wait()
    plsc.subcore_barrier()
    descs = [pltpu.async_copy(ones_v, acc.at[idx_v.at[j]], sem, add=True)
             for j in range(NB)]
    for d in descs:
        d.wait()
    plsc.subcore_barrier()
    pltpu.sync_copy(acc.at[pl.ds(s * SLICE, SLICE)],
                    out_hbm.at[c, pl.ds(s * SLICE, SLICE)])


def _make_prop(D):
    @functools.partial(
        pl.kernel,
        out_type=jax.ShapeDtypeStruct((NC, N_ACC, D), jnp.float32),
        mesh=_mesh,
        scratch_types=[
            pltpu.VMEM((4, 2, EPB), jnp.int32),
            pltpu.VMEM((3, EPB, D), jnp.float32),
            pltpu.VMEM_SHARED((N_ACC, D), jnp.float32),
            [pltpu.SemaphoreType.DMA for _ in range(4)],
            [pltpu.SemaphoreType.DMA for _ in range(3)],
            [pltpu.SemaphoreType.DMA for _ in range(3)],
            pltpu.SemaphoreType.DMA,
        ],
    )
    def _prop(h_hbm, eidx_hbm, out_hbm, eidx, rows, acc,
              isems, gsems, ssems, csem):
        c = lax.axis_index("c")
        s = lax.axis_index("s")
        wid = c * NS + s
        pltpu.async_copy(h_hbm.at[pl.ds(s * SLICE_A, SLICE_A)],
                         acc.at[pl.ds(s * SLICE_A, SLICE_A)], csem).wait()
        plsc.subcore_barrier()

        def iload(t):
            return pltpu.async_copy(eidx_hbm.at[wid, t], eidx.at[t % 4],
                                    isems[t % 4])

        def gat(t):
            return pltpu.async_copy(h_hbm.at[eidx.at[t % 4, 0]],
                                    rows.at[t % 3], gsems[t % 3])

        def sca(t):
            return pltpu.async_copy(rows.at[t % 3],
                                    acc.at[eidx.at[t % 4, 1]],
                                    ssems[t % 3], add=True)

        idd = {t: iload(t) for t in range(min(3, NB))}
        gd = {}
        for t in range(min(2, NB)):
            idd.pop(t).wait()
            gd[t] = gat(t)
        sd = {}
        for t in range(NB):
            gd.pop(t).wait()
            sd[t] = sca(t)
            k = t + 2
            if k < NB:
                if t - 1 in sd:
                    sd.pop(t - 1).wait()
                kk = k + 1
                if kk < NB:
                    idd[kk] = iload(kk)
                if k in idd:
                    idd.pop(k).wait()
                gd[k] = gat(k)
        for t in sorted(sd):
            sd.pop(t).wait()
        plsc.subcore_barrier()
        pltpu.sync_copy(acc.at[pl.ds(s * SLICE_A, SLICE_A)],
                        out_hbm.at[c, pl.ds(s * SLICE_A, SLICE_A)])

    return _prop


_prop128 = _make_prop(HIDDEN)


_R = 512
_G = N_PAD // _R


def _k1_body(p0_ref, p1_ref, x_ref, w_ref, h_ref, dis_ref):
    deg = p0_ref[...] + p1_ref[...] + 1.0
    dis = lax.rsqrt(deg)
    h = jnp.dot(x_ref[...], w_ref[...], preferred_element_type=jnp.float32)
    h_ref[...] = dis * h
    dis_ref[...] = dis


def _k2_body(q0_ref, q1_ref, hp_ref, dis_ref, w_ref, b_ref, out_ref):
    dis = dis_ref[...]
    pre = dis * (q0_ref[...] + q1_ref[...] - hp_ref[...]) + b_ref[...]
    h1 = jnp.maximum(pre, 0.0)
    out_ref[...] = dis * jnp.dot(h1, w_ref[...],
                                 preferred_element_type=jnp.float32)


def _k3_body(q0_ref, q1_ref, hp_ref, dis_ref, b_ref, out_ref):
    o128 = dis_ref[...] * (q0_ref[...] + q1_ref[...] - hp_ref[...])
    o = o128[:, :N_CLASSES] + b_ref[...]
    m = jnp.max(o, axis=1, keepdims=True)
    e = o - m
    lse = jnp.log(jnp.sum(jnp.exp(e), axis=1, keepdims=True))
    out_ref[...] = e - lse


def _col_spec():
    return pl.BlockSpec((_R, 1), lambda i: (i, 0))


def _row_spec(d):
    return pl.BlockSpec((_R, d), lambda i: (i, 0))


def _full_spec(r, d):
    return pl.BlockSpec((r, d), lambda i: (0, 0))


def kernel(x, edge_index, W1, b1, W2, b2):
    src = edge_index[0].astype(jnp.int32)
    dst = edge_index[1].astype(jnp.int32)
    pad_idx = N_NODES + (jnp.arange(E_PAD - N_EDGES, dtype=jnp.int32)
                         % N_DUMMY)
    src3 = jnp.concatenate([src, pad_idx]).reshape(NW, NB, EPB)
    dst3 = jnp.concatenate([dst, pad_idx]).reshape(NW, NB, EPB)
    eidx3 = jnp.stack([src3, dst3], axis=2)
    x_pad = jnp.zeros((N_PAD, IN_FEAT), jnp.float32).at[:N_NODES].set(x)
    b1r = b1.reshape(1, HIDDEN)
    b2r = b2.reshape(1, N_CLASSES)

    degp = _deg_kernel(dst3)
    p0 = degp[0][:, None]
    p1 = degp[1][:, None]

    h1p, dis = pl.pallas_call(
        _k1_body,
        grid=(_G,),
        in_specs=[_col_spec(), _col_spec(), _row_spec(IN_FEAT),
                  _full_spec(IN_FEAT, HIDDEN)],
        out_specs=[_row_spec(HIDDEN), _col_spec()],
        out_shape=[jax.ShapeDtypeStruct((N_PAD, HIDDEN), jnp.float32),
                   jax.ShapeDtypeStruct((N_PAD, 1), jnp.float32)],
    )(p0, p1, x_pad, W1)

    agg1 = _prop128(h1p, eidx3)

    W2p = jnp.zeros((HIDDEN, HIDDEN), jnp.float32).at[:, :N_CLASSES].set(W2)

    h2p = pl.pallas_call(
        _k2_body,
        grid=(_G,),
        in_specs=[_row_spec(HIDDEN), _row_spec(HIDDEN), _row_spec(HIDDEN),
                  _col_spec(), _full_spec(HIDDEN, HIDDEN),
                  _full_spec(1, HIDDEN)],
        out_specs=_row_spec(HIDDEN),
        out_shape=jax.ShapeDtypeStruct((N_PAD, HIDDEN), jnp.float32),
    )(agg1[0], agg1[1], h1p, dis, W2p, b1r)

    agg2 = _prop128(h2p, eidx3)

    out = pl.pallas_call(
        _k3_body,
        grid=(_G,),
        in_specs=[_row_spec(HIDDEN), _row_spec(HIDDEN),
                  _row_spec(HIDDEN), _col_spec(),
                  _full_spec(1, N_CLASSES)],
        out_specs=_row_spec(N_CLASSES),
        out_shape=jax.ShapeDtypeStruct((N_NODES, N_CLASSES), jnp.float32),
    )(agg2[0], agg2[1], h2p, dis, b2r)

    return out


# speedup vs baseline: 1.0841x; 1.0623x over previous
import functools

import jax
import jax.numpy as jnp
from jax import lax
from jax.experimental import pallas as pl
from jax.experimental.pallas import tpu as pltpu
from jax.experimental.pallas import tpu_sc as plsc

N_NODES = 10000
IN_FEAT = 128
HIDDEN = 128
N_CLASSES = 64
N_EDGES = 320000

NC = 2
NS = 16
NW = NC * NS
EPB = 120
NB = 84
PW = NB * EPB
E_PAD = NW * PW
N_PAD = 10240
SLICE = N_PAD // NS
N_ACC = 10112
SLICE_A = N_ACC // NS
N_DUMMY = N_ACC - N_NODES

_mesh = plsc.VectorSubcoreMesh(core_axis_name="c", subcore_axis_name="s")


@functools.partial(
    pl.kernel,
    out_type=jax.ShapeDtypeStruct((NC, N_PAD), jnp.float32),
    mesh=_mesh,
    scratch_types=[
        pltpu.VMEM((NB, 2, EPB), jnp.int32),
        pltpu.VMEM((EPB,), jnp.float32),
        pltpu.VMEM((SLICE,), jnp.float32),
        pltpu.VMEM_SHARED((N_PAD,), jnp.float32),
        pltpu.SemaphoreType.DMA,
    ],
)
def _deg_kernel(dst_hbm, out_hbm, idx_v, ones_v, z_v, acc, sem):
    c = lax.axis_index("c")
    s = lax.axis_index("s")
    wid = c * NS + s
    for i in range(EPB // 16):
        ones_v[pl.ds(i * 16, 16)] = jnp.full((16,), 1.0, jnp.float32)
    if EPB % 16:
        ones_v[pl.ds(EPB - 16, 16)] = jnp.full((16,), 1.0, jnp.float32)
    for i in range(SLICE // 16):
        z_v[pl.ds(i * 16, 16)] = jnp.zeros((16,), jnp.float32)
    pltpu.sync_copy(z_v, acc.at[pl.ds(s * SLICE, SLICE)])
    pltpu.async_copy(dst_hbm.at[wid], idx_v, sem).wait()
    plsc.subcore_barrier()
    descs = [pltpu.async_copy(ones_v, acc.at[idx_v.at[j, 1]], sem, add=True)
             for j in range(NB)]
    for d in descs:
        d.wait()
    plsc.subcore_barrier()
    pltpu.sync_copy(acc.at[pl.ds(s * SLICE, SLICE)],
                    out_hbm.at[c, pl.ds(s * SLICE, SLICE)])


def _make_prop(D):
    @functools.partial(
        pl.kernel,
        out_type=jax.ShapeDtypeStruct((NC, N_ACC, D), jnp.float32),
        mesh=_mesh,
        scratch_types=[
            pltpu.VMEM((4, 2, EPB), jnp.int32),
            pltpu.VMEM((3, EPB, D), jnp.float32),
            pltpu.VMEM_SHARED((N_ACC, D), jnp.float32),
            [pltpu.SemaphoreType.DMA for _ in range(4)],
            [pltpu.SemaphoreType.DMA for _ in range(3)],
            [pltpu.SemaphoreType.DMA for _ in range(3)],
            pltpu.SemaphoreType.DMA,
        ],
    )
    def _prop(h_hbm, eidx_hbm, out_hbm, eidx, rows, acc,
              isems, gsems, ssems, csem):
        c = lax.axis_index("c")
        s = lax.axis_index("s")
        wid = c * NS + s
        pltpu.async_copy(h_hbm.at[pl.ds(s * SLICE_A, SLICE_A)],
                         acc.at[pl.ds(s * SLICE_A, SLICE_A)], csem).wait()
        plsc.subcore_barrier()

        def iload(t):
            return pltpu.async_copy(eidx_hbm.at[wid, t], eidx.at[t % 4],
                                    isems[t % 4])

        def gat(t):
            return pltpu.async_copy(h_hbm.at[eidx.at[t % 4, 0]],
                                    rows.at[t % 3], gsems[t % 3])

        def sca(t):
            return pltpu.async_copy(rows.at[t % 3],
                                    acc.at[eidx.at[t % 4, 1]],
                                    ssems[t % 3], add=True)

        idd = {t: iload(t) for t in range(min(3, NB))}
        gd = {}
        for t in range(min(2, NB)):
            idd.pop(t).wait()
            gd[t] = gat(t)
        sd = {}
        for t in range(NB):
            gd.pop(t).wait()
            sd[t] = sca(t)
            k = t + 2
            if k < NB:
                if t - 1 in sd:
                    sd.pop(t - 1).wait()
                kk = k + 1
                if kk < NB:
                    idd[kk] = iload(kk)
                if k in idd:
                    idd.pop(k).wait()
                gd[k] = gat(k)
        for t in sorted(sd):
            sd.pop(t).wait()
        plsc.subcore_barrier()
        pltpu.sync_copy(acc.at[pl.ds(s * SLICE_A, SLICE_A)],
                        out_hbm.at[c, pl.ds(s * SLICE_A, SLICE_A)])

    return _prop


_prop128 = _make_prop(HIDDEN)


_R = 512
_G = N_PAD // _R


def _k1_body(p0_ref, p1_ref, x_ref, w_ref, h_ref, dis_ref):
    deg = p0_ref[...] + p1_ref[...] + 1.0
    dis = lax.rsqrt(deg)
    h = jnp.dot(x_ref[...], w_ref[...], preferred_element_type=jnp.float32)
    h_ref[...] = dis * h
    dis_ref[...] = dis


def _k2_body(q0_ref, q1_ref, hp_ref, dis_ref, w_ref, b_ref, out_ref):
    dis = dis_ref[...]
    pre = dis * (q0_ref[...] + q1_ref[...] - hp_ref[...]) + b_ref[...]
    h1 = jnp.maximum(pre, 0.0)
    out_ref[...] = dis * jnp.dot(h1, w_ref[...],
                                 preferred_element_type=jnp.float32)


def _k3_body(q0_ref, q1_ref, hp_ref, dis_ref, b_ref, out_ref):
    o128 = dis_ref[...] * (q0_ref[...] + q1_ref[...] - hp_ref[...])
    o = o128[:, :N_CLASSES] + b_ref[...]
    m = jnp.max(o, axis=1, keepdims=True)
    e = o - m
    lse = jnp.log(jnp.sum(jnp.exp(e), axis=1, keepdims=True))
    out_ref[...] = e - lse


def _col_spec():
    return pl.BlockSpec((_R, 1), lambda i: (i, 0))


def _row_spec(d):
    return pl.BlockSpec((_R, d), lambda i: (i, 0))


def _full_spec(r, d):
    return pl.BlockSpec((r, d), lambda i: (0, 0))


def kernel(x, edge_index, W1, b1, W2, b2):
    ei = edge_index.astype(jnp.int32)
    pad_idx = N_NODES + (jnp.arange(E_PAD - N_EDGES, dtype=jnp.int32)
                         % N_DUMMY)
    eflat = jnp.concatenate([ei, jnp.tile(pad_idx, (2, 1))], axis=1)
    eidx3 = eflat.reshape(2, NW, NB, EPB).transpose(1, 2, 0, 3)
    x_pad = jnp.zeros((N_PAD, IN_FEAT), jnp.float32).at[:N_NODES].set(x)
    b1r = b1.reshape(1, HIDDEN)
    b2r = b2.reshape(1, N_CLASSES)

    degp = _deg_kernel(eidx3)
    p0 = degp[0][:, None]
    p1 = degp[1][:, None]

    h1p, dis = pl.pallas_call(
        _k1_body,
        grid=(_G,),
        in_specs=[_col_spec(), _col_spec(), _row_spec(IN_FEAT),
                  _full_spec(IN_FEAT, HIDDEN)],
        out_specs=[_row_spec(HIDDEN), _col_spec()],
        out_shape=[jax.ShapeDtypeStruct((N_PAD, HIDDEN), jnp.float32),
                   jax.ShapeDtypeStruct((N_PAD, 1), jnp.float32)],
    )(p0, p1, x_pad, W1)

    agg1 = _prop128(h1p, eidx3)

    W2p = jnp.zeros((HIDDEN, HIDDEN), jnp.float32).at[:, :N_CLASSES].set(W2)

    h2p = pl.pallas_call(
        _k2_body,
        grid=(_G,),
        in_specs=[_row_spec(HIDDEN), _row_spec(HIDDEN), _row_spec(HIDDEN),
                  _col_spec(), _full_spec(HIDDEN, HIDDEN),
                  _full_spec(1, HIDDEN)],
        out_specs=_row_spec(HIDDEN),
        out_shape=jax.ShapeDtypeStruct((N_PAD, HIDDEN), jnp.float32),
    )(agg1[0], agg1[1], h1p, dis, W2p, b1r)

    agg2 = _prop128(h2p, eidx3)

    out = pl.pallas_call(
        _k3_body,
        grid=(_G,),
        in_specs=[_row_spec(HIDDEN), _row_spec(HIDDEN),
                  _row_spec(HIDDEN), _col_spec(),
                  _full_spec(1, N_CLASSES)],
        out_specs=_row_spec(N_CLASSES),
        out_shape=jax.ShapeDtypeStruct((N_NODES, N_CLASSES), jnp.float32),
    )(agg2[0], agg2[1], h2p, dis, b2r)

    return out


# speedup vs baseline: 1.0876x; 1.0032x over previous
import functools

import jax
import jax.numpy as jnp
from jax import lax
from jax.experimental import pallas as pl
from jax.experimental.pallas import tpu as pltpu
from jax.experimental.pallas import tpu_sc as plsc

N_NODES = 10000
IN_FEAT = 128
HIDDEN = 128
N_CLASSES = 64
N_EDGES = 320000

NC = 2
NS = 16
NW = NC * NS
EPB = 120
NB = 84
PW = NB * EPB
E_PAD = NW * PW
N_PAD = 10240
SLICE = N_PAD // NS
N_ACC = 10112
SLICE_A = N_ACC // NS
N_DUMMY = N_ACC - N_NODES

_mesh = plsc.VectorSubcoreMesh(core_axis_name="c", subcore_axis_name="s")


@functools.partial(
    pl.kernel,
    out_type=jax.ShapeDtypeStruct((NC, N_PAD), jnp.float32),
    mesh=_mesh,
    scratch_types=[
        pltpu.VMEM((NB, 2, EPB), jnp.int32),
        pltpu.VMEM((EPB,), jnp.float32),
        pltpu.VMEM((SLICE,), jnp.float32),
        pltpu.VMEM_SHARED((N_PAD,), jnp.float32),
        pltpu.SemaphoreType.DMA,
    ],
)
def _deg_kernel(dst_hbm, out_hbm, idx_v, ones_v, z_v, acc, sem):
    c = lax.axis_index("c")
    s = lax.axis_index("s")
    wid = c * NS + s
    for i in range(EPB // 16):
        ones_v[pl.ds(i * 16, 16)] = jnp.full((16,), 1.0, jnp.float32)
    if EPB % 16:
        ones_v[pl.ds(EPB - 16, 16)] = jnp.full((16,), 1.0, jnp.float32)
    for i in range(SLICE // 16):
        z_v[pl.ds(i * 16, 16)] = jnp.zeros((16,), jnp.float32)
    pltpu.sync_copy(z_v, acc.at[pl.ds(s * SLICE, SLICE)])
    pltpu.async_copy(dst_hbm.at[wid], idx_v, sem).wait()
    plsc.subcore_barrier()
    descs = [pltpu.async_copy(ones_v, acc.at[idx_v.at[j, 1]], sem, add=True)
             for j in range(NB)]
    for d in descs:
        d.wait()
    plsc.subcore_barrier()
    pltpu.sync_copy(acc.at[pl.ds(s * SLICE, SLICE)],
                    out_hbm.at[c, pl.ds(s * SLICE, SLICE)])


def _make_prop(D):
    @functools.partial(
        pl.kernel,
        out_type=jax.ShapeDtypeStruct((NC, N_ACC, D), jnp.float32),
        mesh=_mesh,
        scratch_types=[
            pltpu.VMEM((4, 2, EPB), jnp.int32),
            pltpu.VMEM((3, EPB, D), jnp.float32),
            pltpu.VMEM_SHARED((N_ACC, D), jnp.float32),
            [pltpu.SemaphoreType.DMA for _ in range(4)],
            [pltpu.SemaphoreType.DMA for _ in range(3)],
            [pltpu.SemaphoreType.DMA for _ in range(3)],
            pltpu.SemaphoreType.DMA,
        ],
    )
    def _prop(h_hbm, eidx_hbm, out_hbm, eidx, rows, acc,
              isems, gsems, ssems, csem):
        c = lax.axis_index("c")
        s = lax.axis_index("s")
        wid = c * NS + s
        pltpu.async_copy(h_hbm.at[pl.ds(s * SLICE_A, SLICE_A)],
                         acc.at[pl.ds(s * SLICE_A, SLICE_A)], csem).wait()
        plsc.subcore_barrier()

        def iload(t):
            return pltpu.async_copy(eidx_hbm.at[wid, t], eidx.at[t % 4],
                                    isems[t % 4])

        def gat(t):
            return pltpu.async_copy(h_hbm.at[eidx.at[t % 4, 0]],
                                    rows.at[t % 3], gsems[t % 3])

        def sca(t):
            return pltpu.async_copy(rows.at[t % 3],
                                    acc.at[eidx.at[t % 4, 1]],
                                    ssems[t % 3], add=True)

        idd = {t: iload(t) for t in range(min(3, NB))}
        gd = {}
        for t in range(min(2, NB)):
            idd.pop(t).wait()
            gd[t] = gat(t)
        sd = {}
        for t in range(NB):
            gd.pop(t).wait()
            sd[t] = sca(t)
            k = t + 2
            if k < NB:
                if t - 1 in sd:
                    sd.pop(t - 1).wait()
                kk = k + 1
                if kk < NB:
                    idd[kk] = iload(kk)
                if k in idd:
                    idd.pop(k).wait()
                gd[k] = gat(k)
        for t in sorted(sd):
            sd.pop(t).wait()
        plsc.subcore_barrier()
        pltpu.sync_copy(acc.at[pl.ds(s * SLICE_A, SLICE_A)],
                        out_hbm.at[c, pl.ds(s * SLICE_A, SLICE_A)])

    return _prop


_prop128 = _make_prop(HIDDEN)


_R = 512
_G = N_PAD // _R


def _k1_body(p0_ref, p1_ref, x_ref, w_ref, h_ref, dis_ref):
    deg = p0_ref[...] + p1_ref[...] + 1.0
    dis = lax.rsqrt(deg)
    h = jnp.dot(x_ref[...], w_ref[...], preferred_element_type=jnp.float32)
    h_ref[...] = dis * h
    dis_ref[...] = dis


def _k2_body(q0_ref, q1_ref, hp_ref, dis_ref, w_ref, b_ref, out_ref):
    dis = dis_ref[...]
    pre = dis * (q0_ref[...] + q1_ref[...] - hp_ref[...]) + b_ref[...]
    h1 = jnp.maximum(pre, 0.0)
    out_ref[...] = dis * jnp.dot(h1, w_ref[...],
                                 preferred_element_type=jnp.float32)


def _k3_body(q0_ref, q1_ref, hp_ref, dis_ref, b_ref, out_ref):
    o128 = dis_ref[...] * (q0_ref[...] + q1_ref[...] - hp_ref[...])
    o = o128[:, :N_CLASSES] + b_ref[...]
    m = jnp.max(o, axis=1, keepdims=True)
    e = o - m
    lse = jnp.log(jnp.sum(jnp.exp(e), axis=1, keepdims=True))
    out_ref[...] = e - lse


def _col_spec():
    return pl.BlockSpec((_R, 1), lambda i: (i, 0))


def _row_spec(d):
    return pl.BlockSpec((_R, d), lambda i: (i, 0))


def _full_spec(r, d):
    return pl.BlockSpec((r, d), lambda i: (0, 0))


def kernel(x, edge_index, W1, b1, W2, b2):
    ei = edge_index.astype(jnp.int32)
    pad_idx = N_NODES + (jnp.arange(E_PAD - N_EDGES, dtype=jnp.int32)
                         % N_DUMMY)
    eflat = jnp.concatenate([ei, jnp.tile(pad_idx, (2, 1))], axis=1)
    eidx3 = eflat.reshape(2, NW, NB, EPB).transpose(1, 2, 0, 3)
    b1r = b1.reshape(1, HIDDEN)
    b2r = b2.reshape(1, N_CLASSES)

    degp = _deg_kernel(eidx3)
    p0 = degp[0][:, None]
    p1 = degp[1][:, None]

    h1p, dis = pl.pallas_call(
        _k1_body,
        grid=(_G,),
        in_specs=[_col_spec(), _col_spec(), _row_spec(IN_FEAT),
                  _full_spec(IN_FEAT, HIDDEN)],
        out_specs=[_row_spec(HIDDEN), _col_spec()],
        out_shape=[jax.ShapeDtypeStruct((N_PAD, HIDDEN), jnp.float32),
                   jax.ShapeDtypeStruct((N_PAD, 1), jnp.float32)],
    )(p0, p1, x, W1)

    agg1 = _prop128(h1p, eidx3)

    W2p = jnp.zeros((HIDDEN, HIDDEN), jnp.float32).at[:, :N_CLASSES].set(W2)

    h2p = pl.pallas_call(
        _k2_body,
        grid=(_G,),
        in_specs=[_row_spec(HIDDEN), _row_spec(HIDDEN), _row_spec(HIDDEN),
                  _col_spec(), _full_spec(HIDDEN, HIDDEN),
                  _full_spec(1, HIDDEN)],
        out_specs=_row_spec(HIDDEN),
        out_shape=jax.ShapeDtypeStruct((N_PAD, HIDDEN), jnp.float32),
    )(agg1[0], agg1[1], h1p, dis, W2p, b1r)

    agg2 = _prop128(h2p, eidx3)

    out = pl.pallas_call(
        _k3_body,
        grid=(_G,),
        in_specs=[_row_spec(HIDDEN), _row_spec(HIDDEN),
                  _row_spec(HIDDEN), _col_spec(),
                  _full_spec(1, N_CLASSES)],
        out_specs=_row_spec(N_CLASSES),
        out_shape=jax.ShapeDtypeStruct((N_NODES, N_CLASSES), jnp.float32),
    )(agg2[0], agg2[1], h2p, dis, b2r)

    return out
